# baseline (speedup 1.0000x reference)
import functools

import jax
import jax.numpy as jnp
from jax import lax
from jax.experimental import pallas as pl
from jax.experimental.pallas import tpu as pltpu
from jax.experimental.pallas import tpu_sc as plsc

NC = 2
NS = 16
NW = NC * NS
CHUNK = 128
NBUF = 4


def _make_sc_gather(vocab, dim, n_chunks):
    mesh = plsc.VectorSubcoreMesh(core_axis_name="c", subcore_axis_name="s")
    total = NW * n_chunks * CHUNK
    n_outer = n_chunks // NBUF

    @functools.partial(
        pl.kernel,
        mesh=mesh,
        out_type=jax.ShapeDtypeStruct((total, dim), jnp.float32),
        compiler_params=pltpu.CompilerParams(use_tc_tiling_on_sc=False),
        scratch_types=(
            [pltpu.VMEM((n_chunks, CHUNK), jnp.int32)]
            + [pltpu.VMEM((CHUNK, dim), jnp.float32) for _ in range(NBUF)]
            + [pltpu.SemaphoreType.DMA for _ in range(2 * NBUF)]
        ),
    )
    def k(table_hbm, idx_hbm, out_hbm, idx_v, *bufs_and_sems):
        rows = bufs_and_sems[:NBUF]
        gsem = bufs_and_sems[NBUF:2 * NBUF]
        psem = bufs_and_sems[2 * NBUF:]
        wid = lax.axis_index("s") * NC + lax.axis_index("c")
        base = wid * (n_chunks * CHUNK)

        pltpu.sync_copy(idx_hbm.at[wid], idx_v)

        def gather(j, b):
            return pltpu.make_async_copy(
                table_hbm.at[idx_v.at[j]], rows[b], gsem[b])

        def put(j, b):
            return pltpu.make_async_copy(
                rows[b], out_hbm.at[pl.ds(base + j * CHUNK, CHUNK)], psem[b])

        for b in range(NBUF):
            gather(b, b).start()

        def outer(g, _):
            for b in range(NBUF):
                j = g * NBUF + b
                gather(j, b).wait()
                put(j, b).start()
                put(j, b).wait()
                gather(j + NBUF, b).start()
            return _

        lax.fori_loop(0, n_outer - 1, outer, None)

        for b in range(NBUF):
            j = (n_outer - 1) * NBUF + b
            gather(j, b).wait()
            put(j, b).start()
            put(j, b).wait()

    return k


def kernel(tokenIndex, e_weights):
    batch, n_fields = tokenIndex.shape
    vocab, dim = e_weights.shape
    total = batch * n_fields
    n_chunks = total // (NW * CHUNK)
    idx3 = tokenIndex.reshape(NW, n_chunks, CHUNK)
    out = _make_sc_gather(vocab, dim, n_chunks)(e_weights, idx3)
    return out.reshape(batch, n_fields, dim)
